# Initial kernel scaffold; baseline (speedup 1.0000x reference)
import jax
import jax.numpy as jnp
from jax.experimental import pallas as pl


def kernel(ref_nor, ref_abn, W_conv, b_conv, W_lin, b_lin, isTrain):
    raise NotImplementedError("write your pallas kernel here")


# speedup vs baseline: 2.9929x; 2.9929x over previous
import functools

import jax
import jax.numpy as jnp
from jax.experimental import pallas as pl


B, T, C_IN, C_OUT = 2, 2048, 2048, 512
PAD = 8
T_TILE = 256
NT = T // T_TILE


def _rapn_kernel(x_ref, w0_ref, w1_ref, w2_ref, bc_ref, wl_ref, bl_ref, out_ref):
    t = pl.program_id(1)
    b = pl.program_id(0)
    ts = pl.multiple_of(t * T_TILE, T_TILE)
    ext = x_ref[0, pl.ds(ts, T_TILE + 2 * PAD), :]
    y0 = jnp.dot(ext, w0_ref[...], preferred_element_type=jnp.float32)
    y1 = jnp.dot(ext, w1_ref[...], preferred_element_type=jnp.float32)
    y2 = jnp.dot(ext, w2_ref[...], preferred_element_type=jnp.float32)
    y = (y0[PAD - 1:PAD - 1 + T_TILE]
         + y1[PAD:PAD + T_TILE]
         + y2[PAD + 1:PAD + 1 + T_TILE])
    y = jnp.maximum(y + bc_ref[...], 0.0)
    logits = jnp.dot(y, wl_ref[...], preferred_element_type=jnp.float32)
    p = jax.nn.sigmoid(logits + bl_ref[0, 0])
    del b
    out_ref[0, pl.ds(ts, T_TILE), :] = p


@functools.partial(jax.jit, static_argnames=())
def _run(x_pad, w0, w1, w2, bc, wl, bl):
    out_t = pl.pallas_call(
        _rapn_kernel,
        grid=(B, NT),
        in_specs=[
            pl.BlockSpec((1, T + 2 * PAD, C_IN), lambda b, t: (b, 0, 0)),
            pl.BlockSpec((C_IN, C_OUT), lambda b, t: (0, 0)),
            pl.BlockSpec((C_IN, C_OUT), lambda b, t: (0, 0)),
            pl.BlockSpec((C_IN, C_OUT), lambda b, t: (0, 0)),
            pl.BlockSpec((1, C_OUT), lambda b, t: (0, 0)),
            pl.BlockSpec((C_OUT, 1), lambda b, t: (0, 0)),
            pl.BlockSpec((1, 1), lambda b, t: (0, 0)),
        ],
        out_specs=pl.BlockSpec((1, T, 1), lambda b, t: (b, 0, 0)),
        out_shape=jax.ShapeDtypeStruct((B, T, 1), jnp.float32),
    )(x_pad, w0, w1, w2, bc, wl, bl)
    return out_t[:, :, 0]


def kernel(ref_nor, ref_abn, W_conv, b_conv, W_lin, b_lin, isTrain):
    del ref_abn, isTrain
    x_pad = jnp.pad(ref_nor, ((0, 0), (PAD, PAD), (0, 0))).astype(jnp.bfloat16)
    w0 = W_conv[:, :, 0].T.astype(jnp.bfloat16)
    w1 = W_conv[:, :, 1].T.astype(jnp.bfloat16)
    w2 = W_conv[:, :, 2].T.astype(jnp.bfloat16)
    bc = b_conv.reshape(1, C_OUT)
    wl = W_lin.reshape(C_OUT, 1).astype(jnp.float32)
    bl = b_lin.reshape(1, 1)
    return _run(x_pad, w0, w1, w2, bc, wl, bl)


# speedup vs baseline: 3.5412x; 1.1832x over previous
import functools

import jax
import jax.numpy as jnp
from jax.experimental import pallas as pl
from jax.experimental.pallas import tpu as pltpu


B, T, C_IN, C_OUT = 2, 2048, 2048, 512
T_TILE = 256
NT = T // T_TILE
EXT = T_TILE + 16


def _rapn_kernel(x_ref, wcat_ref, bc_ref, wl_ref, bl_ref, out_ref):
    t = pl.program_id(1)
    s = t * T_TILE
    es = jnp.where(t == 0, 0, jnp.where(t == NT - 1, T - EXT, s - 8))
    es = pl.multiple_of(es, 8)
    ext = x_ref[0, pl.ds(es, EXT), :].astype(jnp.bfloat16)
    ycat = jnp.dot(ext, wcat_ref[...], preferred_element_type=jnp.float32)
    zrow = jnp.zeros((1, C_OUT), jnp.float32)

    def tail(y):
        y = jnp.maximum(y + bc_ref[...], 0.0)
        logits = jnp.dot(y, wl_ref[...], preferred_element_type=jnp.float32)
        p = jax.nn.sigmoid(logits + bl_ref[0, 0])
        out_ref[0, pl.ds(s, T_TILE), :] = p

    @pl.when(t == 0)
    def _first():
        y = (jnp.concatenate([zrow, ycat[0:T_TILE - 1, 0:C_OUT]], axis=0)
             + ycat[0:T_TILE, C_OUT:2 * C_OUT]
             + ycat[1:T_TILE + 1, 2 * C_OUT:3 * C_OUT])
        tail(y)

    @pl.when(jnp.logical_and(t > 0, t < NT - 1))
    def _mid():
        y = (ycat[7:7 + T_TILE, 0:C_OUT]
             + ycat[8:8 + T_TILE, C_OUT:2 * C_OUT]
             + ycat[9:9 + T_TILE, 2 * C_OUT:3 * C_OUT])
        tail(y)

    @pl.when(t == NT - 1)
    def _last():
        y = (ycat[15:15 + T_TILE, 0:C_OUT]
             + ycat[16:16 + T_TILE, C_OUT:2 * C_OUT]
             + jnp.concatenate([ycat[17:EXT, 2 * C_OUT:3 * C_OUT], zrow], axis=0))
        tail(y)


@functools.partial(jax.jit, static_argnames=())
def _run(x, wcat, bc, wl, bl):
    out_t = pl.pallas_call(
        _rapn_kernel,
        grid=(B, NT),
        in_specs=[
            pl.BlockSpec((1, T, C_IN), lambda b, t: (b, 0, 0)),
            pl.BlockSpec((C_IN, 3 * C_OUT), lambda b, t: (0, 0)),
            pl.BlockSpec((1, C_OUT), lambda b, t: (0, 0)),
            pl.BlockSpec((C_OUT, 1), lambda b, t: (0, 0)),
            pl.BlockSpec((1, 1), lambda b, t: (0, 0)),
        ],
        out_specs=pl.BlockSpec((1, T, 1), lambda b, t: (b, 0, 0)),
        out_shape=jax.ShapeDtypeStruct((B, T, 1), jnp.float32),
        compiler_params=pltpu.CompilerParams(
            vmem_limit_bytes=64 * 1024 * 1024,
        ),
    )(x, wcat, bc, wl, bl)
    return out_t[:, :, 0]


def kernel(ref_nor, ref_abn, W_conv, b_conv, W_lin, b_lin, isTrain):
    del ref_abn, isTrain
    wcat = jnp.concatenate(
        [W_conv[:, :, 0].T, W_conv[:, :, 1].T, W_conv[:, :, 2].T], axis=1
    ).astype(jnp.bfloat16)
    bc = b_conv.reshape(1, C_OUT)
    wl = W_lin.reshape(C_OUT, 1).astype(jnp.float32)
    bl = b_lin.reshape(1, 1)
    return _run(ref_nor, wcat, bc, wl, bl)


# speedup vs baseline: 3.8477x; 1.0865x over previous
import functools

import jax
import jax.numpy as jnp
from jax.experimental import pallas as pl
from jax.experimental.pallas import tpu as pltpu


B, T, C_IN, C_OUT = 2, 2048, 2048, 512
T_TILE = 256
NT = T // T_TILE
EXT = T_TILE + 16
PAD = 8


def _rapn_kernel(x_ref, wcat_ref, bc_ref, wl_ref, bl_ref, out_ref, xp_ref):
    t = pl.program_id(1)

    @pl.when(t == 0)
    def _zero_pad_rows():
        xp_ref[0:PAD, :] = jnp.zeros((PAD, C_IN), jnp.bfloat16)
        xp_ref[T + PAD:T + 2 * PAD, :] = jnp.zeros((PAD, C_IN), jnp.bfloat16)

    @pl.when(t < NT)
    def _cast_block():
        ws = pl.multiple_of(PAD + t * T_TILE, PAD)
        xp_ref[pl.ds(ws, T_TILE), :] = x_ref[0].astype(jnp.bfloat16)

    @pl.when(t > 0)
    def _compute_tile():
        s = pl.multiple_of((t - 1) * T_TILE, T_TILE)
        ext = xp_ref[pl.ds(s, EXT), :]
        ycat = jnp.dot(ext, wcat_ref[...], preferred_element_type=jnp.float32)
        y = (ycat[7:7 + T_TILE, 0:C_OUT]
             + ycat[8:8 + T_TILE, C_OUT:2 * C_OUT]
             + ycat[9:9 + T_TILE, 2 * C_OUT:3 * C_OUT])
        y = jnp.maximum(y + bc_ref[...], 0.0)
        logits = jnp.dot(y, wl_ref[...], preferred_element_type=jnp.float32)
        p = jax.nn.sigmoid(logits + bl_ref[0, 0])
        out_ref[0, pl.ds(s, T_TILE), :] = p


@functools.partial(jax.jit, static_argnames=())
def _run(x, wcat, bc, wl, bl):
    out_t = pl.pallas_call(
        _rapn_kernel,
        grid=(B, NT + 1),
        in_specs=[
            pl.BlockSpec((1, T_TILE, C_IN),
                         lambda b, t: (b, jnp.minimum(t, NT - 1), 0)),
            pl.BlockSpec((C_IN, 3 * C_OUT), lambda b, t: (0, 0)),
            pl.BlockSpec((1, C_OUT), lambda b, t: (0, 0)),
            pl.BlockSpec((C_OUT, 1), lambda b, t: (0, 0)),
            pl.BlockSpec((1, 1), lambda b, t: (0, 0)),
        ],
        out_specs=pl.BlockSpec((1, T, 1), lambda b, t: (b, 0, 0)),
        out_shape=jax.ShapeDtypeStruct((B, T, 1), jnp.float32),
        scratch_shapes=[pltpu.VMEM((T + 2 * PAD, C_IN), jnp.bfloat16)],
        compiler_params=pltpu.CompilerParams(
            vmem_limit_bytes=64 * 1024 * 1024,
        ),
    )(x, wcat, bc, wl, bl)
    return out_t[:, :, 0]


def kernel(ref_nor, ref_abn, W_conv, b_conv, W_lin, b_lin, isTrain):
    del ref_abn, isTrain
    wcat = jnp.concatenate(
        [W_conv[:, :, 0].T, W_conv[:, :, 1].T, W_conv[:, :, 2].T], axis=1
    ).astype(jnp.bfloat16)
    bc = b_conv.reshape(1, C_OUT)
    wl = W_lin.reshape(C_OUT, 1).astype(jnp.float32)
    bl = b_lin.reshape(1, 1)
    return _run(ref_nor, wcat, bc, wl, bl)
